# baseline (speedup 1.0000x reference)
import functools

import jax
import jax.numpy as jnp
from jax import lax
from jax.experimental import pallas as pl
from jax.experimental.pallas import tpu as pltpu
from jax.experimental.pallas import tpu_sc as plsc

NBUF = 4
LANES = 128
D_VALID = 64


def _make_gather(batch: int, seq: int, vocab: int):
    info = plsc.get_sparse_core_info()
    nc, ns = info.num_cores, info.num_subcores
    nw = nc * ns
    assert batch % nw == 0
    rows_per_w = batch // nw
    toks_per_w = rows_per_w * seq

    mesh = plsc.VectorSubcoreMesh(core_axis_name="c", subcore_axis_name="s")

    @functools.partial(
        pl.kernel,
        out_type=jax.ShapeDtypeStruct((batch * seq, LANES), jnp.float32),
        mesh=mesh,
        scratch_types=[
            pltpu.VMEM((toks_per_w,), jnp.int32),
            pltpu.VMEM((NBUF, seq, LANES), jnp.float32),
        ]
        + [pltpu.SemaphoreType.DMA] * NBUF,
    )
    def gather_kernel(idx_hbm, table_hbm, out_hbm, idx_v, rows_v, *sems):
        wid = lax.axis_index("s") * nc + lax.axis_index("c")
        base = wid * rows_per_w
        pltpu.sync_copy(idx_hbm.at[pl.ds(base * seq, toks_per_w)], idx_v)

        def gather_copy(i, buf):
            return pltpu.make_async_copy(
                table_hbm.at[idx_v.at[pl.ds(i * seq, seq)]],
                rows_v.at[buf],
                sems[buf],
            )

        for b in range(NBUF):
            gather_copy(b, b).start()

        def body(g, carry):
            for b in range(NBUF):
                i = NBUF * g + b
                gather_copy(i, b).wait()
                pltpu.sync_copy(
                    rows_v.at[b], out_hbm.at[pl.ds((base + i) * seq, seq)]
                )

                @pl.when(i + NBUF < rows_per_w)
                def _():
                    gather_copy(i + NBUF, b).start()

            return carry

        lax.fori_loop(0, rows_per_w // NBUF, body, 0)

    return gather_kernel


def kernel(x, table):
    b, s = x.shape
    v, d = table.shape
    idx = x.reshape(-1).astype(jnp.int32)
    table_p = jnp.pad(table, ((0, 0), (0, LANES - d)))
    out_p = _make_gather(b, s, v)(idx, table_p)
    return out_p[:, :d].reshape(b, s, d)
